# baseline (speedup 1.0000x reference)
import jax
import jax.numpy as jnp
from jax import lax
from jax.experimental import pallas as pl
from jax.experimental.pallas import tpu as pltpu
from jax.experimental.pallas import tpu_sc as plsc

N = 10000
E = 320000
D = 128
NC = 2
NS = 16
NW = NC * NS
CH = 128
EPT = E // NW
NCHK = 80
NQ = NCHK // 4
EPT_PAD = NCHK * CH
PKW = (NCHK // 2) * CH
NP = 10016
AP = 10240
RPT = AP // NS

_MESH = dict(core_axis_name="c", subcore_axis_name="s",
             num_cores=NC, num_subcores=NS)
_PARAMS = pltpu.CompilerParams(needs_layout_passes=False)


def _tc_matmul_body(x_ref, w_ref, o_ref):
    o_ref[...] = jnp.dot(x_ref[...], w_ref[...],
                         preferred_element_type=jnp.float32)


def _tc_matmul(x, w, interpret=False):
    n, d = x.shape
    k = w.shape[1]
    blk = 400
    grid = n // blk
    return pl.pallas_call(
        _tc_matmul_body,
        interpret=interpret,
        grid=(grid,),
        in_specs=[
            pl.BlockSpec((blk, d), lambda i: (i, 0)),
            pl.BlockSpec((d, k), lambda i: (0, 0)),
        ],
        out_specs=pl.BlockSpec((blk, k), lambda i: (i, 0)),
        out_shape=jax.ShapeDtypeStruct((n, k), jnp.float32),
    )(x, w)


def _unpack_row(pk, r, lo_ref, hi_ref):
    mask = jnp.full((16,), 0xFFFF, jnp.int32)
    sh16 = jnp.full((16,), 16, jnp.int32)
    for k in range(CH // 16):
        w = pk[pl.ds(r * CH + k * 16, 16)]
        lo_ref[pl.ds(k * 16, 16)] = lax.bitwise_and(w, mask)
        hi_ref[pl.ds(k * 16, 16)] = lax.shift_right_logical(w, sh16)


def _sc_agg_body(ys, psrc, pdst, zrow, aggs,
                 psrc_v, pdst_v, srcA0, srcB0, srcA1, srcB1,
                 dstA0, dstB0, dstA1, dstB1, rows0, rows1, acc_sh,
                 gsem0, gsem1, ssem0, ssem1):
    c = lax.axis_index("c")
    s = lax.axis_index("s")

    pltpu.sync_copy(psrc.at[c, s], psrc_v)
    pltpu.sync_copy(pdst.at[c, s], pdst_v)
    pltpu.sync_copy(zrow, acc_sh.at[pl.ds(s * RPT, RPT)])
    plsc.subcore_barrier()

    _unpack_row(psrc_v, 0, srcA0, srcB0)
    _unpack_row(pdst_v, 0, dstA0, dstB0)
    pltpu.async_copy(ys.at[srcA0], rows0, gsem0)

    def wait_g(idx_ref, rows):
        pltpu.make_async_copy(ys.at[idx_ref], rows, gsem0 if rows is rows0
                              else gsem1).wait()

    def wait_s(rows, idx_ref):
        pltpu.make_async_copy(rows, acc_sh.at[idx_ref], ssem0 if rows is rows0
                              else ssem1).wait()

    def quad(qq, carry):
        @pl.when(qq >= 1)
        def _():
            wait_s(rows1, dstB1)
        pltpu.async_copy(ys.at[srcB0], rows1, gsem1)
        _unpack_row(psrc_v, 2 * qq + 1, srcA1, srcB1)
        _unpack_row(pdst_v, 2 * qq + 1, dstA1, dstB1)
        wait_g(srcA0, rows0)
        pltpu.async_copy(rows0, acc_sh.at[dstA0], ssem0, add=True)
        wait_s(rows0, dstA0)
        pltpu.async_copy(ys.at[srcA1], rows0, gsem0)
        wait_g(srcB0, rows1)
        pltpu.async_copy(rows1, acc_sh.at[dstB0], ssem1, add=True)
        wait_s(rows1, dstB0)

        @pl.when(qq + 1 < NQ)
        def _():
            _unpack_row(psrc_v, 2 * qq + 2, srcA0, srcB0)
            _unpack_row(pdst_v, 2 * qq + 2, dstA0, dstB0)
        pltpu.async_copy(ys.at[srcB1], rows1, gsem1)
        wait_g(srcA1, rows0)
        pltpu.async_copy(rows0, acc_sh.at[dstA1], ssem0, add=True)
        wait_s(rows0, dstA1)

        @pl.when(qq + 1 < NQ)
        def _():
            pltpu.async_copy(ys.at[srcA0], rows0, gsem0)
        wait_g(srcB1, rows1)
        pltpu.async_copy(rows1, acc_sh.at[dstB1], ssem1, add=True)
        return carry

    lax.fori_loop(0, NQ, quad, 0, unroll=False)
    wait_s(rows1, dstB1)
    plsc.subcore_barrier()

    pltpu.sync_copy(acc_sh.at[pl.ds(s * RPT, RPT)],
                    aggs.at[pl.ds(c * AP + s * RPT, RPT)])


def _sc_agg(ys, psrc, pdst, zrow, interpret=False):
    idxbuf = pltpu.VMEM((CH,), jnp.int32)
    kfn = pl.kernel(
        _sc_agg_body,
        out_type=[jax.ShapeDtypeStruct((NC * AP, D), jnp.float32)],
        mesh=plsc.VectorSubcoreMesh(**_MESH),
        scratch_types=[
            pltpu.VMEM((PKW,), jnp.int32),
            pltpu.VMEM((PKW,), jnp.int32),
            idxbuf, idxbuf, idxbuf, idxbuf,
            idxbuf, idxbuf, idxbuf, idxbuf,
            pltpu.VMEM((CH, D), jnp.float32),
            pltpu.VMEM((CH, D), jnp.float32),
            pltpu.VMEM_SHARED((AP, D), jnp.float32),
            pltpu.SemaphoreType.DMA, pltpu.SemaphoreType.DMA,
            pltpu.SemaphoreType.DMA, pltpu.SemaphoreType.DMA,
        ],
        compiler_params=_PARAMS,
        interpret=interpret,
    )
    return kfn(ys, psrc, pdst, zrow)[0]


def _sc_cnt_body(dstf, zcnt, cnt_out,
                 dst_v, cnt_v, cbuf, res, cnt_all):
    c = lax.axis_index("c")
    s = lax.axis_index("s")

    pltpu.sync_copy(dstf.at[c, s], dst_v)
    pltpu.sync_copy(zcnt, cnt_v)
    ones = jnp.full((16,), 1.0, jnp.float32)

    def step(i, carry):
        d16 = dst_v[pl.ds(i * 16, 16)]
        plsc.addupdate_scatter(cnt_v, [d16], ones)
        return carry

    lax.fori_loop(0, EPT_PAD // 16, step, 0, unroll=False)

    for t in range(NS):
        pltpu.sync_copy(cnt_v.at[pl.ds(t * RPT, RPT)], cnt_all.at[t, s])
    plsc.subcore_barrier()

    pltpu.sync_copy(cnt_all.at[s], cbuf)

    def rbody(v, carry):
        acc = cbuf[0, pl.ds(v * 16, 16)]
        for t in range(1, NS):
            acc = acc + cbuf[t, pl.ds(v * 16, 16)]
        res[pl.ds(v * 16, 16)] = acc
        return carry

    lax.fori_loop(0, RPT // 16, rbody, 0, unroll=False)
    pltpu.sync_copy(res, cnt_out.at[pl.ds(c * AP + s * RPT, RPT)])


def _sc_cnt(dstf, zcnt, interpret=False):
    kfn = pl.kernel(
        _sc_cnt_body,
        out_type=[jax.ShapeDtypeStruct((NC * AP,), jnp.float32)],
        mesh=plsc.VectorSubcoreMesh(**_MESH),
        scratch_types=[
            pltpu.VMEM((EPT_PAD,), jnp.int32),
            pltpu.VMEM((AP,), jnp.float32),
            pltpu.VMEM((NS, RPT), jnp.float32),
            pltpu.VMEM((RPT,), jnp.float32),
            pltpu.VMEM_SHARED((NS, NS, RPT), jnp.float32),
        ],
        compiler_params=_PARAMS,
        interpret=interpret,
    )
    return kfn(dstf, zcnt)[0]


@jax.jit
def kernel(x, edge_index, Wl1, Wr1, b1, Wl2, Wr2, b2):
    src = edge_index[0]
    dst = edge_index[1]

    pad = jnp.full((NW, EPT_PAD - EPT), N, jnp.int32)
    src_t = jnp.concatenate([src.reshape(NW, EPT), pad], axis=1)
    dst_t = jnp.concatenate([dst.reshape(NW, EPT), pad], axis=1)

    def pack(t):
        v = t.reshape(NW, NCHK // 2, 2, CH)
        return (v[:, :, 0, :] | (v[:, :, 1, :] << 16)).reshape(NC, NS, PKW)

    psrc = pack(src_t)
    pdst = pack(dst_t)
    dstf = dst_t.reshape(NC, NS, EPT_PAD)
    zrow = jnp.zeros((RPT, D), jnp.float32)
    zcnt = jnp.zeros((AP,), jnp.float32)

    wcat1 = jnp.concatenate([Wl1.T, Wr1.T], axis=1)
    wcat2 = jnp.concatenate([Wl2.T, Wr2.T], axis=1)

    def padrows(xl):
        return jnp.pad(xl, ((0, NP - N), (0, 0)))

    cnt2 = _sc_cnt(dstf, zcnt)
    cnt = cnt2[:N] + cnt2[AP:AP + N]
    inv = 1.0 / jnp.maximum(cnt, 1.0)

    o1 = _tc_matmul(x, wcat1)
    xl1, z1 = o1[:, :D], o1[:, D:] + b1
    aggs1 = _sc_agg(padrows(xl1), psrc, pdst, zrow)
    agg1 = aggs1[:N] + aggs1[AP:AP + N]
    h = jax.nn.relu(agg1 * inv[:, None] + z1)

    o2 = _tc_matmul(h, wcat2)
    xl2, z2 = o2[:, :D], o2[:, D:] + b2
    aggs2 = _sc_agg(padrows(xl2), psrc, pdst, zrow)
    agg2 = aggs2[:N] + aggs2[AP:AP + N]
    return agg2 * inv[:, None] + z2
